# baseline (speedup 1.0000x reference)
import functools

import jax
import jax.numpy as jnp
from jax import lax
from jax.experimental import pallas as pl
from jax.experimental.pallas import tpu as pltpu
from jax.experimental.pallas import tpu_sc as plsc

VOCAB = 1_000_000
D = 64
SCALE = 8.0

NC = 2
NS = 16
NW = NC * NS
L = 16

B = 16384 * 26
CHUNK = 128
N_CHUNKS = B // CHUNK
CH_PER_W = N_CHUNKS // NW
ROWS_PER_W = CH_PER_W * CHUNK


def _body(idx_hbm, table_hbm, out_hbm, idx_v, buf, sem):
  wid = lax.axis_index("s") * NC + lax.axis_index("c")
  chunk0 = wid * CH_PER_W
  row0 = wid * ROWS_PER_W

  pltpu.sync_copy(idx_hbm.at[pl.ds(chunk0, CH_PER_W)], idx_v)

  @pl.loop(0, CH_PER_W)
  def _chunk(j):
    pltpu.async_copy(table_hbm.at[idx_v.at[j]], buf, sem).wait()

    @pl.loop(0, CHUNK)
    def _row(r):
      for c in range(D // L):
        buf[r, pl.ds(c * L, L)] = buf[r, pl.ds(c * L, L)] * SCALE

    pltpu.sync_copy(buf, out_hbm.at[pl.ds(row0 + j * CHUNK, CHUNK)])


@jax.jit
def _embed(x_flat, table):
  idx2d = x_flat.reshape(N_CHUNKS, CHUNK)
  mesh = plsc.VectorSubcoreMesh(core_axis_name="c", subcore_axis_name="s")
  out = pl.kernel(
      _body,
      out_type=jax.ShapeDtypeStruct((B, D), jnp.float32),
      mesh=mesh,
      scratch_types=[
          pltpu.VMEM((CH_PER_W, CHUNK), jnp.int32),
          pltpu.VMEM((CHUNK, D), jnp.float32),
          pltpu.SemaphoreType.DMA,
      ],
      compiler_params=pltpu.CompilerParams(use_tc_tiling_on_sc=False),
  )(idx2d, table)
  return out


def kernel(x, table):
  out = _embed(x.reshape(-1).astype(jnp.int32), table)
  return out.reshape(x.shape[0], x.shape[1], D)
